# baseline (device time: 31855 ns/iter reference)
import functools

import jax
import jax.numpy as jnp
from jax import lax
from jax.experimental import pallas as pl
from jax.experimental.pallas import tpu as pltpu

N_DEV = 4
SEQ = 1024
HALO = 128
EXT = SEQ + 2 * HALO
HQ = 8
DH = 128
D = HQ * DH
WINDOW = 128
SCALE = 0.08838834764831843


def kernel(x, Wq, K_ext, V_ext, Wo):
    x2 = x.reshape(SEQ, D)
    K2 = K_ext.reshape(SEQ, D).astype(jnp.bfloat16)
    V2 = V_ext.reshape(SEQ, D).astype(jnp.bfloat16)

    def body(x_ref, wq_ref, k_ref, v_ref, wo_ref, out_ref,
             ext_k, ext_v, ctx_ref, send_sems, recv_sems):
        my = lax.axis_index("i")
        left = (my - 1) % N_DEV
        right = (my + 1) % N_DEV

        barrier = pltpu.get_barrier_semaphore()
        for nbr in (left, right):
            pl.semaphore_signal(barrier, inc=1, device_id=(nbr,),
                                device_id_type=pl.DeviceIdType.MESH)
        pl.semaphore_wait(barrier, 2)

        def rdma(si, src, src_off, ext_ref, dst_off, to):
            return pltpu.make_async_remote_copy(
                src_ref=src.at[pl.ds(src_off, HALO)],
                dst_ref=ext_ref.at[pl.ds(dst_off, HALO)],
                send_sem=send_sems.at[si],
                recv_sem=recv_sems.at[si],
                device_id=(to,),
                device_id_type=pl.DeviceIdType.MESH,
            )

        def rdma_rightward(si, src, ext_ref, to):
            return rdma(si, src, SEQ - HALO, ext_ref, 0, to)

        def rdma_leftward(si, src, ext_ref, to):
            return rdma(si, src, 0, ext_ref, EXT - HALO, to)

        @pl.when(my < N_DEV - 1)
        def _():
            rdma_rightward(0, k_ref, ext_k, right).start()
            rdma_rightward(1, v_ref, ext_v, right).start()

        @pl.when(my > 0)
        def _():
            rdma_leftward(2, k_ref, ext_k, left).start()
            rdma_leftward(3, v_ref, ext_v, left).start()

        ext_k[pl.ds(HALO, SEQ), :] = k_ref[:, :]
        ext_v[pl.ds(HALO, SEQ), :] = v_ref[:, :]

        zeros = jnp.zeros((HALO, D), jnp.bfloat16)

        @pl.when(my == 0)
        def _():
            ext_k[pl.ds(0, HALO), :] = zeros
            ext_v[pl.ds(0, HALO), :] = zeros

        @pl.when(my == N_DEV - 1)
        def _():
            ext_k[pl.ds(EXT - HALO, HALO), :] = zeros
            ext_v[pl.ds(EXT - HALO, HALO), :] = zeros

        q = (jnp.dot(x_ref[:, :].astype(jnp.bfloat16),
                     wq_ref[:, :].astype(jnp.bfloat16),
                     preferred_element_type=jnp.float32)
             * SCALE).astype(jnp.bfloat16)

        QB = 128
        KW = QB + 2 * HALO
        N_QB = SEQ // QB

        r = lax.broadcasted_iota(jnp.int32, (QB, KW), 0)
        c = lax.broadcasted_iota(jnp.int32, (QB, KW), 1)
        base_bias = jnp.where(jnp.abs(r + HALO - c) <= WINDOW, 0.0, -1e9)
        bias_first = jnp.where((my == 0) & (c < HALO), -1e9, base_bias)
        bias_last = jnp.where((my == N_DEV - 1) & (c >= KW - HALO),
                              -1e9, base_bias)

        def attn_block(qb, bias):
            for h in range(HQ):
                qh = q[qb * QB:(qb + 1) * QB, h * DH:(h + 1) * DH]
                kh = ext_k[pl.ds(qb * QB, KW), pl.ds(h * DH, DH)]
                vh = ext_v[pl.ds(qb * QB, KW), pl.ds(h * DH, DH)]
                s = lax.dot_general(
                    qh, kh, (((1,), (1,)), ((), ())),
                    preferred_element_type=jnp.float32,
                ) + bias
                w = jnp.exp(s)
                inv = 1.0 / jnp.sum(w, axis=1, keepdims=True)
                ctx_ref[pl.ds(qb * QB, QB), pl.ds(h * DH, DH)] = (jnp.dot(
                    w.astype(jnp.bfloat16), vh,
                    preferred_element_type=jnp.float32) * inv
                ).astype(jnp.bfloat16)

        for qb in range(1, N_QB - 1):
            attn_block(qb, base_bias)

        @pl.when(my > 0)
        def _():
            rdma_rightward(0, k_ref, ext_k, left).wait_recv()
            rdma_rightward(1, v_ref, ext_v, left).wait_recv()

        attn_block(0, bias_first)

        @pl.when(my < N_DEV - 1)
        def _():
            rdma_leftward(2, k_ref, ext_k, right).wait_recv()
            rdma_leftward(3, v_ref, ext_v, right).wait_recv()

        attn_block(N_QB - 1, bias_last)

        out_ref[:, :] = jnp.dot(ctx_ref[:, :],
                                wo_ref[:, :].astype(jnp.bfloat16),
                                preferred_element_type=jnp.float32)

        @pl.when(my < N_DEV - 1)
        def _():
            rdma_rightward(0, k_ref, ext_k, right).wait_send()
            rdma_rightward(1, v_ref, ext_v, right).wait_send()

        @pl.when(my > 0)
        def _():
            rdma_leftward(2, k_ref, ext_k, left).wait_send()
            rdma_leftward(3, v_ref, ext_v, left).wait_send()

        @functools.partial(pl.run_scoped, sem2=pltpu.SemaphoreType.REGULAR)
        def _(sem2):
            for nbr in (left, right):
                pl.semaphore_signal(sem2, inc=1, device_id=(nbr,),
                                    device_id_type=pl.DeviceIdType.MESH)
            pl.semaphore_wait(sem2, 2)

    out = pl.pallas_call(
        body,
        out_shape=jax.ShapeDtypeStruct((SEQ, D), jnp.float32),
        in_specs=[pl.BlockSpec(memory_space=pltpu.VMEM)] * 5,
        out_specs=pl.BlockSpec(memory_space=pltpu.VMEM),
        scratch_shapes=[
            pltpu.VMEM((EXT, D), jnp.bfloat16),
            pltpu.VMEM((EXT, D), jnp.bfloat16),
            pltpu.VMEM((SEQ, D), jnp.bfloat16),
            pltpu.SemaphoreType.DMA((4,)),
            pltpu.SemaphoreType.DMA((4,)),
        ],
        compiler_params=pltpu.CompilerParams(collective_id=0),
    )(x2, Wq, K2, V2, Wo)
    return out.reshape(1, SEQ, D)


# device time: 27802 ns/iter; 1.1458x vs baseline; 1.1458x over previous
import functools

import jax
import jax.numpy as jnp
from jax import lax
from jax.experimental import pallas as pl
from jax.experimental.pallas import tpu as pltpu

N_DEV = 4
SEQ = 1024
HALO = 128
EXT = SEQ + 2 * HALO
HQ = 8
DH = 128
D = HQ * DH
WINDOW = 128
SCALE = 0.08838834764831843


def kernel(x, Wq, K_ext, V_ext, Wo):
    x2 = x.reshape(SEQ, D)
    K2 = K_ext.reshape(SEQ, D).astype(jnp.bfloat16)
    V2 = V_ext.reshape(SEQ, D).astype(jnp.bfloat16)

    def body(x_ref, wq_ref, k_ref, v_ref, wo_ref, out_ref,
             ext_k, ext_v, ctx_ref, send_sems, recv_sems):
        my = lax.axis_index("i")
        left = (my - 1) % N_DEV
        right = (my + 1) % N_DEV

        barrier = pltpu.get_barrier_semaphore()
        for nbr in (left, right):
            pl.semaphore_signal(barrier, inc=1, device_id=(nbr,),
                                device_id_type=pl.DeviceIdType.MESH)
        pl.semaphore_wait(barrier, 2)

        def rdma(si, src, src_off, ext_ref, dst_off, to):
            return pltpu.make_async_remote_copy(
                src_ref=src.at[pl.ds(src_off, HALO)],
                dst_ref=ext_ref.at[pl.ds(dst_off, HALO)],
                send_sem=send_sems.at[si],
                recv_sem=recv_sems.at[si],
                device_id=(to,),
                device_id_type=pl.DeviceIdType.MESH,
            )

        def rdma_rightward(si, src, ext_ref, to):
            return rdma(si, src, SEQ - HALO, ext_ref, 0, to)

        def rdma_leftward(si, src, ext_ref, to):
            return rdma(si, src, 0, ext_ref, EXT - HALO, to)

        @pl.when(my < N_DEV - 1)
        def _():
            rdma_rightward(0, k_ref, ext_k, right).start()
            rdma_rightward(1, v_ref, ext_v, right).start()

        @pl.when(my > 0)
        def _():
            rdma_leftward(2, k_ref, ext_k, left).start()
            rdma_leftward(3, v_ref, ext_v, left).start()

        ext_k[pl.ds(HALO, SEQ), :] = k_ref[:, :]
        ext_v[pl.ds(HALO, SEQ), :] = v_ref[:, :]

        zeros = jnp.zeros((HALO, D), jnp.bfloat16)

        @pl.when(my == 0)
        def _():
            ext_k[pl.ds(0, HALO), :] = zeros
            ext_v[pl.ds(0, HALO), :] = zeros

        @pl.when(my == N_DEV - 1)
        def _():
            ext_k[pl.ds(EXT - HALO, HALO), :] = zeros
            ext_v[pl.ds(EXT - HALO, HALO), :] = zeros

        q = (jnp.dot(x_ref[:, :].astype(jnp.bfloat16),
                     wq_ref[:, :].astype(jnp.bfloat16),
                     preferred_element_type=jnp.float32)
             * SCALE).astype(jnp.bfloat16)

        QB = 256
        KW = QB + 2 * HALO
        N_QB = SEQ // QB

        r = lax.broadcasted_iota(jnp.int32, (QB, KW), 0)
        c = lax.broadcasted_iota(jnp.int32, (QB, KW), 1)
        base_bias = jnp.where(jnp.abs(r + HALO - c) <= WINDOW, 0.0, -1e9)
        bias_first = jnp.where((my == 0) & (c < HALO), -1e9, base_bias)
        bias_last = jnp.where((my == N_DEV - 1) & (c >= KW - HALO),
                              -1e9, base_bias)

        def attn_block(qb, bias):
            for h in range(HQ):
                qh = q[qb * QB:(qb + 1) * QB, h * DH:(h + 1) * DH]
                kh = ext_k[pl.ds(qb * QB, KW), pl.ds(h * DH, DH)]
                vh = ext_v[pl.ds(qb * QB, KW), pl.ds(h * DH, DH)]
                s = lax.dot_general(
                    qh, kh, (((1,), (1,)), ((), ())),
                    preferred_element_type=jnp.float32,
                ) + bias
                w = jnp.exp(s)
                inv = 1.0 / jnp.sum(w, axis=1, keepdims=True)
                ctx_ref[pl.ds(qb * QB, QB), pl.ds(h * DH, DH)] = (jnp.dot(
                    w.astype(jnp.bfloat16), vh,
                    preferred_element_type=jnp.float32) * inv
                ).astype(jnp.bfloat16)

        attn_block(1, base_bias)
        attn_block(2, base_bias)

        @pl.when(my > 0)
        def _():
            rdma_rightward(0, k_ref, ext_k, left).wait_recv()
            rdma_rightward(1, v_ref, ext_v, left).wait_recv()

        attn_block(0, bias_first)

        @pl.when(my < N_DEV - 1)
        def _():
            rdma_leftward(2, k_ref, ext_k, right).wait_recv()
            rdma_leftward(3, v_ref, ext_v, right).wait_recv()

        attn_block(N_QB - 1, bias_last)

        out_ref[:, :] = jnp.dot(ctx_ref[:, :],
                                wo_ref[:, :].astype(jnp.bfloat16),
                                preferred_element_type=jnp.float32)

        @pl.when(my < N_DEV - 1)
        def _():
            rdma_rightward(0, k_ref, ext_k, right).wait_send()
            rdma_rightward(1, v_ref, ext_v, right).wait_send()

        @pl.when(my > 0)
        def _():
            rdma_leftward(2, k_ref, ext_k, left).wait_send()
            rdma_leftward(3, v_ref, ext_v, left).wait_send()

        @functools.partial(pl.run_scoped, sem2=pltpu.SemaphoreType.REGULAR)
        def _(sem2):
            for nbr in (left, right):
                pl.semaphore_signal(sem2, inc=1, device_id=(nbr,),
                                    device_id_type=pl.DeviceIdType.MESH)
            pl.semaphore_wait(sem2, 2)

    out = pl.pallas_call(
        body,
        out_shape=jax.ShapeDtypeStruct((SEQ, D), jnp.float32),
        in_specs=[pl.BlockSpec(memory_space=pltpu.VMEM)] * 5,
        out_specs=pl.BlockSpec(memory_space=pltpu.VMEM),
        scratch_shapes=[
            pltpu.VMEM((EXT, D), jnp.bfloat16),
            pltpu.VMEM((EXT, D), jnp.bfloat16),
            pltpu.VMEM((SEQ, D), jnp.bfloat16),
            pltpu.SemaphoreType.DMA((4,)),
            pltpu.SemaphoreType.DMA((4,)),
        ],
        compiler_params=pltpu.CompilerParams(collective_id=0),
    )(x2, Wq, K2, V2, Wo)
    return out.reshape(1, SEQ, D)
